# Initial kernel scaffold; baseline (speedup 1.0000x reference)
import functools

import jax
import jax.numpy as jnp
from jax import lax
from jax.experimental import pallas as pl
from jax.experimental.pallas import tpu as pltpu
from jax.experimental.pallas import tpu_sc as plsc

_IDXW = 128
_ROWS_PER_CHUNK = 1024
_SUB = _ROWS_PER_CHUNK // _IDXW


@functools.partial(jax.jit, static_argnums=(2, 3, 4))
def _gather_sc(idx2d, table, n_rows, n_workers, d):
    chunks_per_worker = n_rows // (n_workers * _ROWS_PER_CHUNK)
    idx_rows_per_worker = (n_rows // n_workers) // _IDXW

    mesh = plsc.VectorSubcoreMesh(core_axis_name="c", subcore_axis_name="s")

    @functools.partial(
        pl.kernel,
        mesh=mesh,
        out_type=jax.ShapeDtypeStruct((n_rows, d), jnp.float32),
        scratch_types=[
            pltpu.VMEM((_SUB, _IDXW), jnp.int32),
            pltpu.VMEM((_ROWS_PER_CHUNK, d), jnp.float32),
            pltpu.SemaphoreType.DMA,
        ],
    )
    def k(idx_hbm, table_hbm, out_hbm, idx_v, rows_v, sem):
        nc = 2
        wid = lax.axis_index("s") * nc + lax.axis_index("c")
        idx_row0 = wid * idx_rows_per_worker
        row0 = wid * (idx_rows_per_worker * _IDXW)

        def chunk_body(i, carry):
            irow = idx_row0 + i * _SUB
            base = row0 + i * _ROWS_PER_CHUNK
            pltpu.sync_copy(idx_hbm.at[pl.ds(irow, _SUB)], idx_v)
            copies = []
            for j in range(_SUB):
                copies.append(
                    pltpu.async_copy(
                        table_hbm.at[idx_v.at[j]],
                        rows_v.at[pl.ds(j * _IDXW, _IDXW)],
                        sem,
                    )
                )
            for c in copies:
                c.wait()
            pltpu.sync_copy(rows_v, out_hbm.at[pl.ds(base, _ROWS_PER_CHUNK)])
            return carry

        lax.fori_loop(0, chunks_per_worker, chunk_body, 0)

    return k(idx2d, table)


def kernel(indices, table):
    b, h = indices.shape
    v, d = table.shape
    n = b * h
    info = plsc.get_sparse_core_info()
    n_workers = info.num_cores * info.num_subcores
    idx2d = indices.reshape(n // _IDXW, _IDXW)
    out = _gather_sc(idx2d, table, n, n_workers, d)
    return out.reshape(b, h, d)


# speedup vs baseline: 1.0933x; 1.0933x over previous
import functools

import jax
import jax.numpy as jnp
from jax import lax
from jax.experimental import pallas as pl
from jax.experimental.pallas import tpu as pltpu
from jax.experimental.pallas import tpu_sc as plsc

_IDXW = 128
_ROWS_PER_CHUNK = 1024
_SUB = _ROWS_PER_CHUNK // _IDXW


@functools.partial(jax.jit, static_argnums=(2, 3, 4))
def _gather_sc(idx2d, table, n_rows, n_workers, d):
    chunks_per_worker = n_rows // (n_workers * _ROWS_PER_CHUNK)
    idx_rows_per_worker = (n_rows // n_workers) // _IDXW

    mesh = plsc.VectorSubcoreMesh(core_axis_name="c", subcore_axis_name="s")

    @functools.partial(
        pl.kernel,
        mesh=mesh,
        compiler_params=pltpu.CompilerParams(use_tc_tiling_on_sc=False),
        out_type=jax.ShapeDtypeStruct((n_rows, d), jnp.float32),
        scratch_types=[
            pltpu.VMEM((_SUB, _IDXW), jnp.int32),
            pltpu.VMEM((_ROWS_PER_CHUNK, d), jnp.float32),
            pltpu.SemaphoreType.DMA,
        ],
    )
    def k(idx_hbm, table_hbm, out_hbm, idx_v, rows_v, sem):
        nc = 2
        wid = lax.axis_index("s") * nc + lax.axis_index("c")
        idx_row0 = wid * idx_rows_per_worker
        row0 = wid * (idx_rows_per_worker * _IDXW)

        def chunk_body(i, carry):
            irow = idx_row0 + i * _SUB
            base = row0 + i * _ROWS_PER_CHUNK
            pltpu.sync_copy(idx_hbm.at[pl.ds(irow, _SUB)], idx_v)
            copies = []
            for j in range(_SUB):
                copies.append(
                    pltpu.async_copy(
                        table_hbm.at[idx_v.at[j]],
                        rows_v.at[pl.ds(j * _IDXW, _IDXW)],
                        sem,
                    )
                )
            for c in copies:
                c.wait()
            pltpu.sync_copy(rows_v, out_hbm.at[pl.ds(base, _ROWS_PER_CHUNK)])
            return carry

        lax.fori_loop(0, chunks_per_worker, chunk_body, 0)

    return k(idx2d, table)


def kernel(indices, table):
    b, h = indices.shape
    v, d = table.shape
    n = b * h
    info = plsc.get_sparse_core_info()
    n_workers = info.num_cores * info.num_subcores
    idx2d = indices.reshape(n // _IDXW, _IDXW)
    out = _gather_sc(idx2d, table, n, n_workers, d)
    return out.reshape(b, h, d)


# speedup vs baseline: 1.7383x; 1.5899x over previous
import functools

import jax
import jax.numpy as jnp
from jax import lax
from jax.experimental import pallas as pl
from jax.experimental.pallas import tpu as pltpu
from jax.experimental.pallas import tpu_sc as plsc

_CHUNK_B = 16


@functools.partial(jax.jit, static_argnums=(2,))
def _gather_sc(indices, table, n_workers):
    b, h = indices.shape
    _, d = table.shape
    b_per_w = b // n_workers
    chunks_per_worker = b_per_w // _CHUNK_B

    mesh = plsc.VectorSubcoreMesh(core_axis_name="c", subcore_axis_name="s")

    @functools.partial(
        pl.kernel,
        mesh=mesh,
        compiler_params=pltpu.CompilerParams(use_tc_tiling_on_sc=False),
        out_type=jax.ShapeDtypeStruct((b, h, d), jnp.float32),
        scratch_types=[
            pltpu.VMEM((_CHUNK_B, h), jnp.int32),
            pltpu.VMEM((_CHUNK_B, h, d), jnp.float32),
            pltpu.SemaphoreType.DMA,
        ],
    )
    def k(idx_hbm, table_hbm, out_hbm, idx_v, rows_v, sem):
        nc = 2
        wid = lax.axis_index("s") * nc + lax.axis_index("c")
        b0 = wid * b_per_w

        def chunk_body(i, carry):
            base = b0 + i * _CHUNK_B
            pltpu.sync_copy(idx_hbm.at[pl.ds(base, _CHUNK_B)], idx_v)
            copies = []
            for j in range(_CHUNK_B):
                copies.append(
                    pltpu.async_copy(
                        table_hbm.at[idx_v.at[j]],
                        rows_v.at[j],
                        sem,
                    )
                )
            for c in copies:
                c.wait()
            pltpu.sync_copy(rows_v, out_hbm.at[pl.ds(base, _CHUNK_B)])
            return carry

        lax.fori_loop(0, chunks_per_worker, chunk_body, 0)

    return k(indices, table)


def kernel(indices, table):
    info = plsc.get_sparse_core_info()
    n_workers = info.num_cores * info.num_subcores
    return _gather_sc(indices, table, n_workers)
